# baseline (speedup 1.0000x reference)
import functools

import jax
import jax.numpy as jnp
from jax import lax
from jax.experimental import pallas as pl
from jax.experimental.pallas import tpu as pltpu
from jax.experimental.pallas import tpu_sc as plsc

_VOTE_THRESH = 0.65
_MAX_DET = 100
_NSUB = 16
_NCORE = 2
_NCLASS = 80
_LANES = 16


def _sc_body(x1h, y1h, x2h, y2h, sch, labh, part_out, head_out,
             vx1, vy1, vx2, vy2, vsc, vlab, varea,
             pub_vm, pub2, partf, headf, pub_sh, *, shard, nvec):
    c = lax.axis_index("c")
    s = lax.axis_index("s")
    base = s * shard
    lanes = lax.iota(jnp.int32, _LANES)
    zf = jnp.zeros((_LANES,), jnp.float32)

    pltpu.sync_copy(x1h.at[pl.ds(base, shard)], vx1)
    pltpu.sync_copy(y1h.at[pl.ds(base, shard)], vy1)
    pltpu.sync_copy(x2h.at[pl.ds(base, shard)], vx2)
    pltpu.sync_copy(y2h.at[pl.ds(base, shard)], vy2)
    pltpu.sync_copy(sch.at[pl.ds(base, shard)], vsc)
    pltpu.sync_copy(labh.at[pl.ds(base, shard)], vlab)

    def maxstep(j, mv):
        return jnp.maximum(mv, jnp.maximum(vx2[pl.ds(j * _LANES, _LANES)],
                                           vy2[pl.ds(j * _LANES, _LANES)]))
    mvec = lax.fori_loop(0, nvec, maxstep, zf)
    mloc = jnp.max(mvec)
    pub_vm[pl.ds(0, _LANES)] = jnp.where(lanes == 0, mloc, 0.0)
    pltpu.sync_copy(pub_vm, pub_sh.at[s])
    plsc.subcore_barrier()
    pltpu.sync_copy(pub_sh, pub2)
    plsc.subcore_barrier()
    zcol = jnp.zeros((_LANES,), jnp.int32)
    mcoord = jnp.max(plsc.load_gather(pub2, [lanes, zcol])) + 1.0

    lo = (c * (_NCLASS // _NCORE)).astype(jnp.float32)
    hi = lo + float(_NCLASS // _NCORE)

    def offstep(j, _):
        sl = pl.ds(j * _LANES, _LANES)
        lb = vlab[sl]
        off = lb * mcoord
        a = vx1[sl] + off
        b = vy1[sl] + off
        d = vx2[sl] + off
        e = vy2[sl] + off
        vx1[sl] = a
        vy1[sl] = b
        vx2[sl] = d
        vy2[sl] = e
        varea[sl] = (d - a) * (e - b)
        inr = (lb >= lo) & (lb < hi)
        vsc[sl] = jnp.where(inr, vsc[sl], -1.0)
        return 0
    lax.fori_loop(0, nvec, offstep, 0)

    def amstep(j, carry):
        cv, ci = carry
        sl = pl.ds(j * _LANES, _LANES)
        val = vsc[sl]
        idx = j * _LANES + lanes
        upd = val > cv
        return (jnp.where(upd, val, cv), jnp.where(upd, idx, ci))
    cv0 = jnp.full((_LANES,), -1.0, jnp.float32)
    cand = lax.fori_loop(0, nvec, amstep, (cv0, zcol))

    big = jnp.int32(10 ** 9)
    bigf = jnp.float32(10 ** 9)

    def cluster(k, carry):
        cv, ci = carry
        mlc = jnp.max(cv)
        lidx = jnp.min(jnp.where(cv == mlc, ci, big))
        iv = jnp.broadcast_to(lidx, (_LANES,))
        ghx1 = plsc.load_gather(vx1, [iv])
        ghy1 = plsc.load_gather(vy1, [iv])
        ghx2 = plsc.load_gather(vx2, [iv])
        ghy2 = plsc.load_gather(vy2, [iv])
        ghlb = plsc.load_gather(vlab, [iv])
        gidxf = (base + lidx).astype(jnp.float32)
        row = jnp.where(lanes == 0, mlc, 0.0)
        row = jnp.where(lanes == 1, gidxf, row)
        row = jnp.where(lanes == 2, ghx1, row)
        row = jnp.where(lanes == 3, ghy1, row)
        row = jnp.where(lanes == 4, ghx2, row)
        row = jnp.where(lanes == 5, ghy2, row)
        row = jnp.where(lanes == 6, ghlb, row)
        pub_vm[pl.ds(0, _LANES)] = row
        pltpu.sync_copy(pub_vm, pub_sh.at[s])
        plsc.subcore_barrier()
        pltpu.sync_copy(pub_sh, pub2)
        plsc.subcore_barrier()
        col = lambda q: plsc.load_gather(pub2, [lanes, jnp.full((_LANES,), q, jnp.int32)])
        scs = col(0)
        m = jnp.max(scs)
        any_left = m >= 0.0
        idxs = col(1)
        gidx = jnp.min(jnp.where(scs == m, idxs, bigf))
        wsel = (scs == m) & (idxs == gidx)
        hx1 = jnp.sum(jnp.where(wsel, col(2), 0.0))
        hy1 = jnp.sum(jnp.where(wsel, col(3), 0.0))
        hx2 = jnp.sum(jnp.where(wsel, col(4), 0.0))
        hy2 = jnp.sum(jnp.where(wsel, col(5), 0.0))
        hlb = jnp.sum(jnp.where(wsel, col(6), 0.0))
        harea = (hx2 - hx1) * (hy2 - hy1)

        def sweep(j, sc_carry):
            p1, p2, p3, p4, p5, scv, sci = sc_carry
            sl = pl.ds(j * _LANES, _LANES)
            a = vx1[sl]
            b = vy1[sl]
            d = vx2[sl]
            e = vy2[sl]
            sj = vsc[sl]
            w = jnp.maximum(jnp.minimum(hx2, d) - jnp.maximum(hx1, a), 0.0)
            h = jnp.maximum(jnp.minimum(hy2, e) - jnp.maximum(hy1, b), 0.0)
            inter = w * h
            iou = inter / (harea + varea[sl] - inter)
            merge = (iou >= _VOTE_THRESH) & (sj >= 0.0) & any_left
            mw = jnp.where(merge, sj, 0.0)
            p1 = p1 + mw * a
            p2 = p2 + mw * b
            p3 = p3 + mw * d
            p4 = p4 + mw * e
            p5 = p5 + mw
            sj = jnp.where(merge, -1.0, sj)
            vsc[sl] = sj
            idx = j * _LANES + lanes
            upd = sj > scv
            return (p1, p2, p3, p4, p5,
                    jnp.where(upd, sj, scv), jnp.where(upd, idx, sci))

        init = (zf, zf, zf, zf, zf, cv0, zcol)
        p1, p2, p3, p4, p5, ncv, nci = lax.fori_loop(0, nvec, sweep, init)

        prow = jnp.where(lanes == 0, jnp.sum(p1), 0.0)
        prow = jnp.where(lanes == 1, jnp.sum(p2), prow)
        prow = jnp.where(lanes == 2, jnp.sum(p3), prow)
        prow = jnp.where(lanes == 3, jnp.sum(p4), prow)
        prow = jnp.where(lanes == 4, jnp.sum(p5), prow)
        partf[pl.ds(k * _LANES, _LANES)] = prow

        @pl.when(s == 0)
        def _():
            hrow = jnp.where(lanes == 0, m, 0.0)
            hrow = jnp.where(lanes == 1, gidx, hrow)
            hrow = jnp.where(lanes == 2, hlb, hrow)
            hrow = jnp.where(lanes == 3, jnp.where(any_left, 1.0, 0.0), hrow)
            hrow = jnp.where(lanes == 4, mcoord, hrow)
            headf[pl.ds(k * _LANES, _LANES)] = hrow

        return (ncv, nci)

    lax.fori_loop(0, _MAX_DET, cluster, cand)

    pltpu.sync_copy(partf, part_out.at[c * _NSUB + s])

    @pl.when(s == 0)
    def _():
        pltpu.sync_copy(headf, head_out.at[c])


def _sc_greedy(x1, y1, x2, y2, sc, lab, *, shard, nvec):
    mesh = plsc.VectorSubcoreMesh(core_axis_name="c", subcore_axis_name="s")
    f = pl.kernel(
        functools.partial(_sc_body, shard=shard, nvec=nvec),
        out_type=(
            jax.ShapeDtypeStruct((_NCORE * _NSUB, _MAX_DET * _LANES), jnp.float32),
            jax.ShapeDtypeStruct((_NCORE, _MAX_DET * _LANES), jnp.float32),
        ),
        mesh=mesh,
        compiler_params=pltpu.CompilerParams(needs_layout_passes=False),
        scratch_types=[
            pltpu.VMEM((shard,), jnp.float32),
            pltpu.VMEM((shard,), jnp.float32),
            pltpu.VMEM((shard,), jnp.float32),
            pltpu.VMEM((shard,), jnp.float32),
            pltpu.VMEM((shard,), jnp.float32),
            pltpu.VMEM((shard,), jnp.float32),
            pltpu.VMEM((shard,), jnp.float32),
            pltpu.VMEM((128,), jnp.float32),
            pltpu.VMEM((_LANES, 128), jnp.float32),
            pltpu.VMEM((_MAX_DET * _LANES,), jnp.float32),
            pltpu.VMEM((_MAX_DET * _LANES,), jnp.float32),
            pltpu.VMEM_SHARED((_LANES, 128), jnp.float32),
        ],
    )
    return f(x1, y1, x2, y2, sc, lab)


def _merge_body(part_ref, head_ref, out_ref):
    ncand = _NCORE * _MAX_DET
    P = part_ref[...]
    S0 = jnp.sum(P[0:_NSUB], axis=0, keepdims=True)
    S1 = jnp.sum(P[_NSUB:2 * _NSUB], axis=0, keepdims=True)
    S = jnp.concatenate([S0, S1], axis=0)
    H = head_ref[...]
    sco = H[:, 0:1]
    idx = H[:, 1:2]
    labc = H[:, 2:3]
    vld = H[:, 3:4]
    mcoord = jnp.max(H[:, 4:5])
    rows = lax.broadcasted_iota(jnp.int32, (ncand, 1), 0)
    srow = lax.broadcasted_iota(jnp.int32, S.shape, 0)
    slane = lax.broadcasted_iota(jnp.int32, S.shape, 1)
    lane = lax.broadcasted_iota(jnp.int32, (1, 128), 1)
    zrow = jnp.zeros((1, 128), jnp.float32)
    bigf = jnp.float32(10 ** 9)
    big = jnp.int32(10 ** 9)

    def step(i, carry):
        alive, ax1, ay1, ax2, ay2, asc, alab, aval = carry
        ms = jnp.where(alive > 0.0, sco, -1.0)
        m = jnp.max(ms)
        any_left = m >= 0.0
        gidx = jnp.min(jnp.where((ms == m) & (alive > 0.0), idx, bigf))
        rsel = (ms == m) & (idx == gidx) & (alive > 0.0)
        r = jnp.min(jnp.where(rsel, rows, big))
        cstar = r // _MAX_DET
        kstar = r - cstar * _MAX_DET
        hlab = jnp.sum(jnp.where(rsel, labc, 0.0))
        sel = lambda q: jnp.sum(jnp.where(
            (srow == cstar) & (slane == kstar * _LANES + q), S, 0.0))
        sw = sel(4)
        denom = jnp.where(any_left, sw, 1.0)
        off = hlab * mcoord
        km = lane == i
        ax1 = jnp.where(km, jnp.where(any_left, sel(0) / denom - off, 0.0), ax1)
        ay1 = jnp.where(km, jnp.where(any_left, sel(1) / denom - off, 0.0), ay1)
        ax2 = jnp.where(km, jnp.where(any_left, sel(2) / denom - off, 0.0), ax2)
        ay2 = jnp.where(km, jnp.where(any_left, sel(3) / denom - off, 0.0), ay2)
        asc = jnp.where(km, jnp.where(any_left, m, 0.0), asc)
        alab = jnp.where(km, jnp.where(any_left, hlab, -1.0), alab)
        aval = jnp.where(km & any_left, 1.0, aval)
        alive = jnp.where(rows == r, 0.0, alive)
        return (alive, ax1, ay1, ax2, ay2, asc, alab, aval)

    init = (vld, zrow, zrow, zrow, zrow, zrow, zrow, zrow)
    carry = lax.fori_loop(0, _MAX_DET, step, init)
    _, ax1, ay1, ax2, ay2, asc, alab, _ = carry
    out_ref[...] = jnp.concatenate(
        [ax1, ay1, ax2, ay2, asc, alab, zrow, zrow], axis=0)


def kernel(boxes, scores, labels):
    n = boxes.shape[0]
    shard = -(-n // (_NSUB * _LANES)) * _LANES
    nvec = shard // _LANES
    p = _NSUB * shard - n
    labf = labels.astype(jnp.float32)

    def pad(a, v):
        return jnp.pad(a, (0, p), constant_values=v)

    part, head = _sc_greedy(
        pad(boxes[:, 0], 0.0), pad(boxes[:, 1], 0.0),
        pad(boxes[:, 2], 0.0), pad(boxes[:, 3], 0.0),
        pad(scores, -1.0), pad(labf, 0.0),
        shard=shard, nvec=nvec)

    out = pl.pallas_call(
        _merge_body,
        out_shape=jax.ShapeDtypeStruct((8, 128), jnp.float32),
    )(part, head.reshape(_NCORE * _MAX_DET, _LANES))
    out_boxes = out[0:4, :_MAX_DET].T
    out_scores = out[4, :_MAX_DET]
    out_labels = out[5, :_MAX_DET]
    return out_boxes, out_scores, out_labels
